# Initial kernel scaffold; baseline (speedup 1.0000x reference)
import functools

import jax
import jax.numpy as jnp
from jax import lax
from jax.experimental import pallas as pl
from jax.experimental.pallas import tpu as pltpu
from jax.experimental.pallas import tpu_sc as plsc

BN_EPS = 1e-5

NC = 2
NS = 16
NW = NC * NS


def _mm_body(x_ref, w_ref, y_ref):
    y_ref[...] = jnp.dot(x_ref[...], w_ref[0], preferred_element_type=jnp.float32)


def _transform_features(x, W):
    n, c_in = x.shape
    k, _, c_out = W.shape
    bn = 2000
    nb = n // bn
    return pl.pallas_call(
        _mm_body,
        grid=(k, nb),
        in_specs=[
            pl.BlockSpec((bn, c_in), lambda i, j: (j, 0)),
            pl.BlockSpec((1, c_in, c_out), lambda i, j: (i, 0, 0)),
        ],
        out_specs=pl.BlockSpec((bn, c_out), lambda i, j: (i * nb + j, 0)),
        out_shape=jax.ShapeDtypeStruct((k * n, c_out), jnp.float32),
    )(x, W)


def _make_sc_edge_kernel(n, c, ch, b):
    mesh = plsc.VectorSubcoreMesh(
        core_axis_name="c", subcore_axis_name="s", num_cores=NC, num_subcores=NS
    )
    rows_per_sub = n // NS

    @functools.partial(
        pl.kernel,
        mesh=mesh,
        out_type=jax.ShapeDtypeStruct((NC, n, c), jnp.float32),
        scratch_types=[
            pltpu.VMEM_SHARED((n, c), jnp.float32),
            pltpu.VMEM((ch, b), jnp.int32),
            pltpu.VMEM((ch, b), jnp.int32),
            pltpu.VMEM((b, c), jnp.float32),
            pltpu.VMEM((b, c), jnp.float32),
            pltpu.SemaphoreType.DMA,
            pltpu.SemaphoreType.DMA,
        ],
    )
    def sc_kernel(y_hbm, gidx_hbm, dst_hbm, zeros_hbm, part_hbm,
                  acc, gidx_v, dst_v, rows_a, rows_b, sem_a, sem_b):
        cid = lax.axis_index("c")
        sid = lax.axis_index("s")
        wid = cid * NS + sid

        pltpu.sync_copy(zeros_hbm, acc.at[pl.ds(sid * rows_per_sub, rows_per_sub)])
        pltpu.sync_copy(gidx_hbm.at[wid], gidx_v)
        pltpu.sync_copy(dst_hbm.at[wid], dst_v)
        plsc.subcore_barrier()

        def pair_body(p, carry):
            g = p * 2
            d_a = pltpu.async_copy(y_hbm.at[gidx_v.at[g]], rows_a, sem_a)
            d_b = pltpu.async_copy(y_hbm.at[gidx_v.at[g + 1]], rows_b, sem_b)
            d_a.wait()
            pltpu.sync_copy(rows_a, acc.at[dst_v.at[g]], add=True)
            d_b.wait()
            pltpu.sync_copy(rows_b, acc.at[dst_v.at[g + 1]], add=True)
            return carry

        lax.fori_loop(0, ch // 2, pair_body, 0)
        plsc.subcore_barrier()

        pltpu.sync_copy(
            acc.at[pl.ds(sid * rows_per_sub, rows_per_sub)],
            part_hbm.at[cid, pl.ds(sid * rows_per_sub, rows_per_sub)],
        )

    return sc_kernel


def _combine_stats_body(nb, n, p_ref, s_ref, mean_ref, var_ref, acc_s, acc_q):
    j = pl.program_id(0)
    blk = p_ref[0] + p_ref[1]
    s_ref[...] = blk
    ps = jnp.sum(blk, axis=0, keepdims=True)
    pq = jnp.sum(blk * blk, axis=0, keepdims=True)

    @pl.when(j == 0)
    def _():
        acc_s[...] = ps
        acc_q[...] = pq

    @pl.when(j > 0)
    def _():
        acc_s[...] += ps
        acc_q[...] += pq

    @pl.when(j == nb - 1)
    def _():
        m = acc_s[...] / n
        mean_ref[...] = m
        var_ref[...] = acc_q[...] / n - m * m


def _combine_stats(part):
    _, n, c = part.shape
    br = 2000
    nb = n // br
    return pl.pallas_call(
        functools.partial(_combine_stats_body, nb, n),
        grid=(nb,),
        in_specs=[pl.BlockSpec((2, br, c), lambda j: (0, j, 0))],
        out_specs=[
            pl.BlockSpec((br, c), lambda j: (j, 0)),
            pl.BlockSpec((1, c), lambda j: (0, 0)),
            pl.BlockSpec((1, c), lambda j: (0, 0)),
        ],
        out_shape=[
            jax.ShapeDtypeStruct((n, c), jnp.float32),
            jax.ShapeDtypeStruct((1, c), jnp.float32),
            jax.ShapeDtypeStruct((1, c), jnp.float32),
        ],
        scratch_shapes=[
            pltpu.VMEM((1, c), jnp.float32),
            pltpu.VMEM((1, c), jnp.float32),
        ],
    )(part)


def _bn_body(s_ref, mean_ref, var_ref, g_ref, b_ref, o_ref):
    inv = lax.rsqrt(var_ref[...] + BN_EPS)
    o_ref[...] = (s_ref[...] - mean_ref[...]) * (inv * g_ref[...]) + b_ref[...]


def _bn_apply(s, mean, var, gamma, beta):
    n, c = s.shape
    br = 2000
    nb = n // br
    vec = pl.BlockSpec((1, c), lambda j: (0, 0))
    return pl.pallas_call(
        _bn_body,
        grid=(nb,),
        in_specs=[pl.BlockSpec((br, c), lambda j: (j, 0)), vec, vec, vec, vec],
        out_specs=pl.BlockSpec((br, c), lambda j: (j, 0)),
        out_shape=jax.ShapeDtypeStruct((n, c), jnp.float32),
    )(s, mean, var, gamma, beta)


@jax.jit
def kernel(x, edge_index, kernel_offsets, W, bn_gamma, bn_beta):
    n, c_in = x.shape
    k, _, c_out = W.shape
    e = edge_index.shape[1]

    y = _transform_features(x, W)

    src = edge_index[0]
    dst = edge_index[1]
    gidx = kernel_offsets * n + src

    b = 125
    ch = e // (NW * b)
    gidx_t = gidx.reshape(NW, ch, b)
    dst_t = dst.reshape(NW, ch, b)
    zeros = jnp.zeros((n // NS, c_out), jnp.float32)

    part = _make_sc_edge_kernel(n, c_out, ch, b)(y, gidx_t, dst_t, zeros)

    s, mean, var = _combine_stats(part)
    return _bn_apply(s, mean, var, bn_gamma.reshape(1, c_out),
                     bn_beta.reshape(1, c_out))


# speedup vs baseline: 3.1729x; 3.1729x over previous
import functools

import jax
import jax.numpy as jnp
from jax import lax
from jax.experimental import pallas as pl
from jax.experimental.pallas import tpu as pltpu
from jax.experimental.pallas import tpu_sc as plsc

BN_EPS = 1e-5

NC = 2
NS = 16
NW = NC * NS


def _mm_body(x_ref, w_ref, y_ref):
    y_ref[...] = jnp.dot(x_ref[...], w_ref[0], preferred_element_type=jnp.float32)


def _transform_features(x, W):
    n, c_in = x.shape
    k, _, c_out = W.shape
    bn = 2000
    nb = n // bn
    return pl.pallas_call(
        _mm_body,
        grid=(k, nb),
        in_specs=[
            pl.BlockSpec((bn, c_in), lambda i, j: (j, 0)),
            pl.BlockSpec((1, c_in, c_out), lambda i, j: (i, 0, 0)),
        ],
        out_specs=pl.BlockSpec((bn, c_out), lambda i, j: (i * nb + j, 0)),
        out_shape=jax.ShapeDtypeStruct((k * n, c_out), jnp.float32),
    )(x, W)


def _make_sc_edge_kernel(n, c, e, ch, b):
    mesh = plsc.VectorSubcoreMesh(
        core_axis_name="c", subcore_axis_name="s", num_cores=NC, num_subcores=NS
    )
    rows_per_sub = n // NS
    e_per_w = e // NW

    @functools.partial(
        pl.kernel,
        mesh=mesh,
        out_type=jax.ShapeDtypeStruct((NC, NS, rows_per_sub, c), jnp.float32),
        scratch_types=[
            pltpu.VMEM_SHARED((n, c), jnp.float32),
            pltpu.VMEM((e_per_w,), jnp.int32),
            pltpu.VMEM((b,), jnp.int32),
            pltpu.VMEM((b,), jnp.int32),
            pltpu.VMEM((b, c), jnp.float32),
            pltpu.VMEM((b, c), jnp.float32),
            pltpu.SemaphoreType.DMA,
            pltpu.SemaphoreType.DMA,
            pltpu.SemaphoreType.DMA,
            pltpu.SemaphoreType.DMA,
        ],
    )
    def sc_kernel(y_hbm, gidx_hbm, dst_hbm, zeros_hbm, part_hbm,
                  acc, gidx_v, dst_a, dst_b, rows_a, rows_b,
                  sem_ra, sem_rb, sem_da, sem_db):
        cid = lax.axis_index("c")
        sid = lax.axis_index("s")
        wid = cid * NS + sid
        base = wid * e_per_w

        pltpu.sync_copy(zeros_hbm, acc.at[pl.ds(sid * rows_per_sub, rows_per_sub)])
        pltpu.sync_copy(gidx_hbm.at[pl.ds(base, e_per_w)], gidx_v)
        plsc.subcore_barrier()

        def chunk(g, dst_buf, rows_buf, sem_r, sem_d):
            d_d = pltpu.async_copy(dst_hbm.at[pl.ds(base + g * b, b)], dst_buf, sem_d)
            d_r = pltpu.async_copy(y_hbm.at[gidx_v.at[pl.ds(g * b, b)]], rows_buf, sem_r)
            return d_d, d_r

        def drain(pair):
            d_d, d_r = pair
            d_r.wait()
            d_d.wait()

        def pair_body(p, carry):
            g = p * 2
            in_a = chunk(g, dst_a, rows_a, sem_ra, sem_da)
            in_b = chunk(g + 1, dst_b, rows_b, sem_rb, sem_db)
            drain(in_a)
            pltpu.sync_copy(rows_a, acc.at[dst_a], add=True)
            drain(in_b)
            pltpu.sync_copy(rows_b, acc.at[dst_b], add=True)
            return carry

        lax.fori_loop(0, ch // 2, pair_body, 0)
        if ch % 2:
            in_t = chunk(ch - 1, dst_a, rows_a, sem_ra, sem_da)
            drain(in_t)
            pltpu.sync_copy(rows_a, acc.at[dst_a], add=True)
        plsc.subcore_barrier()

        pltpu.sync_copy(
            acc.at[pl.ds(sid * rows_per_sub, rows_per_sub)],
            part_hbm.at[cid, sid],
        )

    return sc_kernel


def _combine_stats_body(nb, n, p_ref, s_ref, mean_ref, var_ref, acc_s, acc_q):
    j = pl.program_id(0)
    blk = p_ref[0] + p_ref[1]
    s_ref[...] = blk
    ps = jnp.sum(blk, axis=0, keepdims=True)
    pq = jnp.sum(blk * blk, axis=0, keepdims=True)

    @pl.when(j == 0)
    def _():
        acc_s[...] = ps
        acc_q[...] = pq

    @pl.when(j > 0)
    def _():
        acc_s[...] += ps
        acc_q[...] += pq

    @pl.when(j == nb - 1)
    def _():
        m = acc_s[...] / n
        mean_ref[...] = m
        var_ref[...] = acc_q[...] / n - m * m


def _combine_stats(part):
    _, n, c = part.shape
    br = 2000
    nb = n // br
    return pl.pallas_call(
        functools.partial(_combine_stats_body, nb, n),
        grid=(nb,),
        in_specs=[pl.BlockSpec((2, br, c), lambda j: (0, j, 0))],
        out_specs=[
            pl.BlockSpec((br, c), lambda j: (j, 0)),
            pl.BlockSpec((1, c), lambda j: (0, 0)),
            pl.BlockSpec((1, c), lambda j: (0, 0)),
        ],
        out_shape=[
            jax.ShapeDtypeStruct((n, c), jnp.float32),
            jax.ShapeDtypeStruct((1, c), jnp.float32),
            jax.ShapeDtypeStruct((1, c), jnp.float32),
        ],
        scratch_shapes=[
            pltpu.VMEM((1, c), jnp.float32),
            pltpu.VMEM((1, c), jnp.float32),
        ],
    )(part)


def _bn_body(s_ref, mean_ref, var_ref, g_ref, b_ref, o_ref):
    inv = lax.rsqrt(var_ref[...] + BN_EPS)
    o_ref[...] = (s_ref[...] - mean_ref[...]) * (inv * g_ref[...]) + b_ref[...]


def _bn_apply(s, mean, var, gamma, beta):
    n, c = s.shape
    br = 2000
    nb = n // br
    vec = pl.BlockSpec((1, c), lambda j: (0, 0))
    return pl.pallas_call(
        _bn_body,
        grid=(nb,),
        in_specs=[pl.BlockSpec((br, c), lambda j: (j, 0)), vec, vec, vec, vec],
        out_specs=pl.BlockSpec((br, c), lambda j: (j, 0)),
        out_shape=jax.ShapeDtypeStruct((n, c), jnp.float32),
    )(s, mean, var, gamma, beta)


@jax.jit
def kernel(x, edge_index, kernel_offsets, W, bn_gamma, bn_beta):
    n, c_in = x.shape
    k, _, c_out = W.shape
    e = edge_index.shape[1]

    y = _transform_features(x, W)

    src = edge_index[0]
    dst = edge_index[1]
    gidx = kernel_offsets * n + src

    b = 80
    ch = e // (NW * b)
    zeros = jnp.zeros((n // NS, c_out), jnp.float32)

    part = _make_sc_edge_kernel(n, c_out, e, ch, b)(y, gidx, dst, zeros)
    part = part.reshape(NC, n, c_out)

    s, mean, var = _combine_stats(part)
    return _bn_apply(s, mean, var, bn_gamma.reshape(1, c_out),
                     bn_beta.reshape(1, c_out))


# speedup vs baseline: 4.4068x; 1.3889x over previous
import functools

import jax
import jax.numpy as jnp
from jax import lax
from jax.experimental import pallas as pl
from jax.experimental.pallas import tpu as pltpu
from jax.experimental.pallas import tpu_sc as plsc

BN_EPS = 1e-5

NC = 2
NS = 16
NW = NC * NS


def _mm_body(x_ref, w_ref, y_ref):
    y_ref[...] = jnp.dot(x_ref[...], w_ref[0], preferred_element_type=jnp.float32)


def _transform_features(x, W):
    n, c_in = x.shape
    k, _, c_out = W.shape
    return pl.pallas_call(
        _mm_body,
        grid=(k,),
        in_specs=[
            pl.BlockSpec((n, c_in), lambda i: (0, 0)),
            pl.BlockSpec((1, c_in, c_out), lambda i: (i, 0, 0)),
        ],
        out_specs=pl.BlockSpec((n, c_out), lambda i: (i, 0)),
        out_shape=jax.ShapeDtypeStruct((k * n, c_out), jnp.float32),
    )(x, W)


def _make_sc_edge_kernel(n, c, e, ch, b):
    mesh = plsc.VectorSubcoreMesh(
        core_axis_name="c", subcore_axis_name="s", num_cores=NC, num_subcores=NS
    )
    rows_per_sub = n // NS
    e_per_w = e // NW

    @functools.partial(
        pl.kernel,
        mesh=mesh,
        out_type=jax.ShapeDtypeStruct((NC, NS, rows_per_sub, c), jnp.float32),
        scratch_types=[
            pltpu.VMEM_SHARED((n, c), jnp.float32),
            pltpu.VMEM((e_per_w,), jnp.int32),
            pltpu.VMEM((b,), jnp.int32),
            pltpu.VMEM((b,), jnp.int32),
            pltpu.VMEM((b, c), jnp.float32),
            pltpu.VMEM((b, c), jnp.float32),
            pltpu.SemaphoreType.DMA,
            pltpu.SemaphoreType.DMA,
            pltpu.SemaphoreType.DMA,
            pltpu.SemaphoreType.DMA,
            pltpu.SemaphoreType.DMA,
            pltpu.SemaphoreType.DMA,
        ],
    )
    def sc_kernel(y_hbm, gidx_hbm, dst_hbm, zeros_hbm, part_hbm,
                  acc, gidx_v, dst_a, dst_b, rows_a, rows_b,
                  sem_ra, sem_rb, sem_da, sem_db, sem_sa, sem_sb):
        cid = lax.axis_index("c")
        sid = lax.axis_index("s")
        wid = cid * NS + sid
        base = wid * e_per_w

        pltpu.sync_copy(zeros_hbm, acc.at[pl.ds(sid * rows_per_sub, rows_per_sub)])
        pltpu.sync_copy(gidx_hbm.at[pl.ds(base, e_per_w)], gidx_v)
        plsc.subcore_barrier()

        def chunk(g, dst_buf, rows_buf, sem_r, sem_d):
            d_d = pltpu.async_copy(dst_hbm.at[pl.ds(base + g * b, b)], dst_buf, sem_d)
            d_r = pltpu.async_copy(y_hbm.at[gidx_v.at[pl.ds(g * b, b)]], rows_buf, sem_r)
            return d_d, d_r

        def drain(pair):
            d_d, d_r = pair
            d_r.wait()
            d_d.wait()

        def pair_body(p, carry):
            g = p * 2
            in_a = chunk(g, dst_a, rows_a, sem_ra, sem_da)
            in_b = chunk(g + 1, dst_b, rows_b, sem_rb, sem_db)
            drain(in_a)
            s_a = pltpu.async_copy(rows_a, acc.at[dst_a], sem_sa, add=True)
            drain(in_b)
            s_b = pltpu.async_copy(rows_b, acc.at[dst_b], sem_sb, add=True)
            s_a.wait()
            s_b.wait()
            return carry

        lax.fori_loop(0, ch // 2, pair_body, 0)
        if ch % 2:
            in_t = chunk(ch - 1, dst_a, rows_a, sem_ra, sem_da)
            drain(in_t)
            pltpu.sync_copy(rows_a, acc.at[dst_a], add=True)
        plsc.subcore_barrier()

        pltpu.sync_copy(
            acc.at[pl.ds(sid * rows_per_sub, rows_per_sub)],
            part_hbm.at[cid, sid],
        )

    return sc_kernel


def _combine_stats_body(nb, n, p_ref, s_ref, mean_ref, var_ref, acc_s, acc_q):
    j = pl.program_id(0)
    blk = p_ref[0] + p_ref[1]
    s_ref[...] = blk
    ps = jnp.sum(blk, axis=0, keepdims=True)
    pq = jnp.sum(blk * blk, axis=0, keepdims=True)

    @pl.when(j == 0)
    def _():
        acc_s[...] = ps
        acc_q[...] = pq

    @pl.when(j > 0)
    def _():
        acc_s[...] += ps
        acc_q[...] += pq

    @pl.when(j == nb - 1)
    def _():
        m = acc_s[...] / n
        mean_ref[...] = m
        var_ref[...] = acc_q[...] / n - m * m


def _combine_stats(part):
    _, n, c = part.shape
    br = 2000
    nb = n // br
    return pl.pallas_call(
        functools.partial(_combine_stats_body, nb, n),
        grid=(nb,),
        in_specs=[pl.BlockSpec((2, br, c), lambda j: (0, j, 0))],
        out_specs=[
            pl.BlockSpec((br, c), lambda j: (j, 0)),
            pl.BlockSpec((1, c), lambda j: (0, 0)),
            pl.BlockSpec((1, c), lambda j: (0, 0)),
        ],
        out_shape=[
            jax.ShapeDtypeStruct((n, c), jnp.float32),
            jax.ShapeDtypeStruct((1, c), jnp.float32),
            jax.ShapeDtypeStruct((1, c), jnp.float32),
        ],
        scratch_shapes=[
            pltpu.VMEM((1, c), jnp.float32),
            pltpu.VMEM((1, c), jnp.float32),
        ],
    )(part)


def _bn_body(s_ref, mean_ref, var_ref, g_ref, b_ref, o_ref):
    inv = lax.rsqrt(var_ref[...] + BN_EPS)
    o_ref[...] = (s_ref[...] - mean_ref[...]) * (inv * g_ref[...]) + b_ref[...]


def _bn_apply(s, mean, var, gamma, beta):
    n, c = s.shape
    br = 2000
    nb = n // br
    vec = pl.BlockSpec((1, c), lambda j: (0, 0))
    return pl.pallas_call(
        _bn_body,
        grid=(nb,),
        in_specs=[pl.BlockSpec((br, c), lambda j: (j, 0)), vec, vec, vec, vec],
        out_specs=pl.BlockSpec((br, c), lambda j: (j, 0)),
        out_shape=jax.ShapeDtypeStruct((n, c), jnp.float32),
    )(s, mean, var, gamma, beta)


@jax.jit
def kernel(x, edge_index, kernel_offsets, W, bn_gamma, bn_beta):
    n, c_in = x.shape
    k, _, c_out = W.shape
    e = edge_index.shape[1]

    y = _transform_features(x, W)

    src = edge_index[0]
    dst = edge_index[1]
    gidx = kernel_offsets * n + src

    b = 80
    ch = e // (NW * b)
    zeros = jnp.zeros((n // NS, c_out), jnp.float32)

    part = _make_sc_edge_kernel(n, c_out, e, ch, b)(y, gidx, dst, zeros)
    part = part.reshape(NC, n, c_out)

    s, mean, var = _combine_stats(part)
    return _bn_apply(s, mean, var, bn_gamma.reshape(1, c_out),
                     bn_beta.reshape(1, c_out))


# speedup vs baseline: 4.6873x; 1.0637x over previous
import functools

import jax
import jax.numpy as jnp
from jax import lax
from jax.experimental import pallas as pl
from jax.experimental.pallas import tpu as pltpu
from jax.experimental.pallas import tpu_sc as plsc

BN_EPS = 1e-5

NC = 2
NS = 16
NW = NC * NS


def _mm_body(x_ref, w_ref, y_ref):
    y_ref[...] = jnp.dot(x_ref[...], w_ref[0], preferred_element_type=jnp.float32)


def _transform_features(x, W):
    n, c_in = x.shape
    k, _, c_out = W.shape
    return pl.pallas_call(
        _mm_body,
        grid=(k,),
        in_specs=[
            pl.BlockSpec((n, c_in), lambda i: (0, 0)),
            pl.BlockSpec((1, c_in, c_out), lambda i: (i, 0, 0)),
        ],
        out_specs=pl.BlockSpec((n, c_out), lambda i: (i, 0)),
        out_shape=jax.ShapeDtypeStruct((k * n, c_out), jnp.float32),
    )(x, W)


def _make_sc_edge_kernel(n, c, e, ch, b):
    mesh = plsc.VectorSubcoreMesh(
        core_axis_name="c", subcore_axis_name="s", num_cores=NC, num_subcores=NS
    )
    rows_per_sub = n // NS
    e_per_w = e // NW

    @functools.partial(
        pl.kernel,
        mesh=mesh,
        out_type=jax.ShapeDtypeStruct((NC, NS, rows_per_sub, c), jnp.float32),
        scratch_types=[
            pltpu.VMEM_SHARED((n, c), jnp.float32),
            pltpu.VMEM((e_per_w,), jnp.int32),
            pltpu.VMEM((b,), jnp.int32),
            pltpu.VMEM((b,), jnp.int32),
            pltpu.VMEM((b,), jnp.int32),
            pltpu.VMEM((b, c), jnp.float32),
            pltpu.VMEM((b, c), jnp.float32),
            pltpu.VMEM((b, c), jnp.float32),
            pltpu.SemaphoreType.DMA,
            pltpu.SemaphoreType.DMA,
            pltpu.SemaphoreType.DMA,
            pltpu.SemaphoreType.DMA,
            pltpu.SemaphoreType.DMA,
            pltpu.SemaphoreType.DMA,
            pltpu.SemaphoreType.DMA,
            pltpu.SemaphoreType.DMA,
            pltpu.SemaphoreType.DMA,
        ],
    )
    def sc_kernel(y_hbm, gidx_hbm, dst_hbm, zeros_hbm, part_hbm,
                  acc, gidx_v, dst_a, dst_b, dst_c, rows_a, rows_b, rows_c,
                  sem_ra, sem_rb, sem_rc, sem_da, sem_db, sem_dc,
                  sem_sa, sem_sb, sem_sc):
        cid = lax.axis_index("c")
        sid = lax.axis_index("s")
        wid = cid * NS + sid
        base = wid * e_per_w

        pltpu.sync_copy(zeros_hbm, acc.at[pl.ds(sid * rows_per_sub, rows_per_sub)])
        pltpu.sync_copy(gidx_hbm.at[pl.ds(base, e_per_w)], gidx_v)
        plsc.subcore_barrier()

        def chunk(g, dst_buf, rows_buf, sem_r, sem_d):
            d_d = pltpu.async_copy(dst_hbm.at[pl.ds(base + g * b, b)], dst_buf, sem_d)
            d_r = pltpu.async_copy(y_hbm.at[gidx_v.at[pl.ds(g * b, b)]], rows_buf, sem_r)
            return d_d, d_r

        def drain(pair):
            d_d, d_r = pair
            d_r.wait()
            d_d.wait()

        def triple_body(p, carry):
            g = p * 3
            in_a = chunk(g, dst_a, rows_a, sem_ra, sem_da)
            in_b = chunk(g + 1, dst_b, rows_b, sem_rb, sem_db)
            in_c = chunk(g + 2, dst_c, rows_c, sem_rc, sem_dc)
            drain(in_a)
            s_a = pltpu.async_copy(rows_a, acc.at[dst_a], sem_sa, add=True)
            drain(in_b)
            s_b = pltpu.async_copy(rows_b, acc.at[dst_b], sem_sb, add=True)
            drain(in_c)
            s_c = pltpu.async_copy(rows_c, acc.at[dst_c], sem_sc, add=True)
            s_a.wait()
            s_b.wait()
            s_c.wait()
            return carry

        lax.fori_loop(0, ch // 3, triple_body, 0)
        for g in range((ch // 3) * 3, ch):
            in_t = chunk(g, dst_a, rows_a, sem_ra, sem_da)
            drain(in_t)
            pltpu.sync_copy(rows_a, acc.at[dst_a], add=True)
        plsc.subcore_barrier()

        pltpu.sync_copy(
            acc.at[pl.ds(sid * rows_per_sub, rows_per_sub)],
            part_hbm.at[cid, sid],
        )

    return sc_kernel


def _combine_stats_body(nb, n, p_ref, s_ref, mean_ref, var_ref, acc_s, acc_q):
    j = pl.program_id(0)
    blk = p_ref[0] + p_ref[1]
    s_ref[...] = blk
    ps = jnp.sum(blk, axis=0, keepdims=True)
    pq = jnp.sum(blk * blk, axis=0, keepdims=True)

    @pl.when(j == 0)
    def _():
        acc_s[...] = ps
        acc_q[...] = pq

    @pl.when(j > 0)
    def _():
        acc_s[...] += ps
        acc_q[...] += pq

    @pl.when(j == nb - 1)
    def _():
        m = acc_s[...] / n
        mean_ref[...] = m
        var_ref[...] = acc_q[...] / n - m * m


def _combine_stats(part):
    _, n, c = part.shape
    br = 2000
    nb = n // br
    return pl.pallas_call(
        functools.partial(_combine_stats_body, nb, n),
        grid=(nb,),
        in_specs=[pl.BlockSpec((2, br, c), lambda j: (0, j, 0))],
        out_specs=[
            pl.BlockSpec((br, c), lambda j: (j, 0)),
            pl.BlockSpec((1, c), lambda j: (0, 0)),
            pl.BlockSpec((1, c), lambda j: (0, 0)),
        ],
        out_shape=[
            jax.ShapeDtypeStruct((n, c), jnp.float32),
            jax.ShapeDtypeStruct((1, c), jnp.float32),
            jax.ShapeDtypeStruct((1, c), jnp.float32),
        ],
        scratch_shapes=[
            pltpu.VMEM((1, c), jnp.float32),
            pltpu.VMEM((1, c), jnp.float32),
        ],
    )(part)


def _bn_body(s_ref, mean_ref, var_ref, g_ref, b_ref, o_ref):
    inv = lax.rsqrt(var_ref[...] + BN_EPS)
    o_ref[...] = (s_ref[...] - mean_ref[...]) * (inv * g_ref[...]) + b_ref[...]


def _bn_apply(s, mean, var, gamma, beta):
    n, c = s.shape
    br = 2000
    nb = n // br
    vec = pl.BlockSpec((1, c), lambda j: (0, 0))
    return pl.pallas_call(
        _bn_body,
        grid=(nb,),
        in_specs=[pl.BlockSpec((br, c), lambda j: (j, 0)), vec, vec, vec, vec],
        out_specs=pl.BlockSpec((br, c), lambda j: (j, 0)),
        out_shape=jax.ShapeDtypeStruct((n, c), jnp.float32),
    )(s, mean, var, gamma, beta)


@jax.jit
def kernel(x, edge_index, kernel_offsets, W, bn_gamma, bn_beta):
    n, c_in = x.shape
    k, _, c_out = W.shape
    e = edge_index.shape[1]

    y = _transform_features(x, W)

    src = edge_index[0]
    dst = edge_index[1]
    gidx = kernel_offsets * n + src

    b = 80
    ch = e // (NW * b)
    zeros = jnp.zeros((n // NS, c_out), jnp.float32)

    part = _make_sc_edge_kernel(n, c_out, e, ch, b)(y, gidx, dst, zeros)
    part = part.reshape(NC, n, c_out)

    s, mean, var = _combine_stats(part)
    return _bn_apply(s, mean, var, bn_gamma.reshape(1, c_out),
                     bn_beta.reshape(1, c_out))


# speedup vs baseline: 4.8202x; 1.0284x over previous
import functools

import jax
import jax.numpy as jnp
from jax import lax
from jax.experimental import pallas as pl
from jax.experimental.pallas import tpu as pltpu
from jax.experimental.pallas import tpu_sc as plsc

BN_EPS = 1e-5

NC = 2
NS = 16
NW = NC * NS


def _mm_body(x_ref, w_ref, y_ref):
    y_ref[...] = jnp.dot(x_ref[...], w_ref[0], preferred_element_type=jnp.float32)


def _transform_features(x, W):
    n, c_in = x.shape
    k, _, c_out = W.shape
    return pl.pallas_call(
        _mm_body,
        grid=(k,),
        in_specs=[
            pl.BlockSpec((n, c_in), lambda i: (0, 0)),
            pl.BlockSpec((1, c_in, c_out), lambda i: (i, 0, 0)),
        ],
        out_specs=pl.BlockSpec((n, c_out), lambda i: (i, 0)),
        out_shape=jax.ShapeDtypeStruct((k * n, c_out), jnp.float32),
    )(x, W)


def _make_sc_edge_kernel(n, c, e, ch, b):
    mesh = plsc.VectorSubcoreMesh(
        core_axis_name="c", subcore_axis_name="s", num_cores=NC, num_subcores=NS
    )
    rows_per_sub = n // NS
    e_per_w = e // NW

    @functools.partial(
        pl.kernel,
        mesh=mesh,
        out_type=jax.ShapeDtypeStruct((NC, NS, rows_per_sub, c), jnp.float32),
        scratch_types=[
            pltpu.VMEM_SHARED((n, c), jnp.float32),
            pltpu.VMEM((e_per_w,), jnp.int32),
            pltpu.VMEM((b,), jnp.int32),
            pltpu.VMEM((b,), jnp.int32),
            pltpu.VMEM((b,), jnp.int32),
            pltpu.VMEM((b, c), jnp.float32),
            pltpu.VMEM((b, c), jnp.float32),
            pltpu.VMEM((b, c), jnp.float32),
            pltpu.SemaphoreType.DMA,
            pltpu.SemaphoreType.DMA,
            pltpu.SemaphoreType.DMA,
            pltpu.SemaphoreType.DMA,
            pltpu.SemaphoreType.DMA,
            pltpu.SemaphoreType.DMA,
            pltpu.SemaphoreType.DMA,
            pltpu.SemaphoreType.DMA,
            pltpu.SemaphoreType.DMA,
        ],
    )
    def sc_kernel(y_hbm, gidx_hbm, dst_hbm, zeros_hbm, part_hbm,
                  acc, gidx_v, dst_a, dst_b, dst_c, rows_a, rows_b, rows_c,
                  sem_ra, sem_rb, sem_rc, sem_da, sem_db, sem_dc,
                  sem_sa, sem_sb, sem_sc):
        cid = lax.axis_index("c")
        sid = lax.axis_index("s")
        wid = cid * NS + sid
        base = wid * e_per_w

        pltpu.sync_copy(gidx_hbm.at[pl.ds(base, e_per_w)], gidx_v)

        def chunk(g, dst_buf, rows_buf, sem_r, sem_d):
            d_d = pltpu.async_copy(dst_hbm.at[pl.ds(base + g * b, b)], dst_buf, sem_d)
            d_r = pltpu.async_copy(y_hbm.at[gidx_v.at[pl.ds(g * b, b)]], rows_buf, sem_r)
            return d_d, d_r

        def drain(pair):
            d_d, d_r = pair
            d_r.wait()
            d_d.wait()

        in_a0 = chunk(0, dst_a, rows_a, sem_ra, sem_da)
        in_b0 = chunk(1, dst_b, rows_b, sem_rb, sem_db)
        in_c0 = chunk(2, dst_c, rows_c, sem_rc, sem_dc)
        pltpu.sync_copy(zeros_hbm, acc.at[pl.ds(sid * rows_per_sub, rows_per_sub)])
        plsc.subcore_barrier()

        def scatter_triple(in_a, in_b, in_c):
            drain(in_a)
            s_a = pltpu.async_copy(rows_a, acc.at[dst_a], sem_sa, add=True)
            drain(in_b)
            s_b = pltpu.async_copy(rows_b, acc.at[dst_b], sem_sb, add=True)
            drain(in_c)
            s_c = pltpu.async_copy(rows_c, acc.at[dst_c], sem_sc, add=True)
            s_a.wait()
            s_b.wait()
            s_c.wait()

        scatter_triple(in_a0, in_b0, in_c0)

        def triple_body(p, carry):
            g = p * 3
            in_a = chunk(g, dst_a, rows_a, sem_ra, sem_da)
            in_b = chunk(g + 1, dst_b, rows_b, sem_rb, sem_db)
            in_c = chunk(g + 2, dst_c, rows_c, sem_rc, sem_dc)
            scatter_triple(in_a, in_b, in_c)
            return carry

        lax.fori_loop(1, ch // 3, triple_body, 0)
        for g in range((ch // 3) * 3, ch):
            in_t = chunk(g, dst_a, rows_a, sem_ra, sem_da)
            drain(in_t)
            pltpu.sync_copy(rows_a, acc.at[dst_a], add=True)
        plsc.subcore_barrier()

        pltpu.sync_copy(
            acc.at[pl.ds(sid * rows_per_sub, rows_per_sub)],
            part_hbm.at[cid, sid],
        )

    return sc_kernel


def _combine_bn_body(n, p_ref, g_ref, b_ref, o_ref):
    s = p_ref[0] + p_ref[1]
    mean = jnp.sum(s, axis=0, keepdims=True) / n
    var = jnp.sum(s * s, axis=0, keepdims=True) / n - mean * mean
    scale = lax.rsqrt(var + BN_EPS) * g_ref[...]
    o_ref[...] = (s - mean) * scale + b_ref[...]


def _combine_bn(part, gamma, beta):
    _, n, c = part.shape
    return pl.pallas_call(
        functools.partial(_combine_bn_body, n),
        in_specs=[
            pl.BlockSpec((2, n, c), lambda: (0, 0, 0)),
            pl.BlockSpec((1, c), lambda: (0, 0)),
            pl.BlockSpec((1, c), lambda: (0, 0)),
        ],
        out_specs=pl.BlockSpec((n, c), lambda: (0, 0)),
        out_shape=jax.ShapeDtypeStruct((n, c), jnp.float32),
    )(part, gamma, beta)


@jax.jit
def kernel(x, edge_index, kernel_offsets, W, bn_gamma, bn_beta):
    n, c_in = x.shape
    k, _, c_out = W.shape
    e = edge_index.shape[1]

    y = _transform_features(x, W)

    src = edge_index[0]
    dst = edge_index[1]
    gidx = kernel_offsets * n + src

    b = 80
    ch = e // (NW * b)
    zeros = jnp.zeros((n // NS, c_out), jnp.float32)

    part = _make_sc_edge_kernel(n, c_out, e, ch, b)(y, gidx, dst, zeros)
    part = part.reshape(NC, n, c_out)

    return _combine_bn(part, bn_gamma.reshape(1, c_out),
                       bn_beta.reshape(1, c_out))


# speedup vs baseline: 5.3096x; 1.1015x over previous
import functools

import jax
import jax.numpy as jnp
from jax import lax
from jax.experimental import pallas as pl
from jax.experimental.pallas import tpu as pltpu
from jax.experimental.pallas import tpu_sc as plsc

BN_EPS = 1e-5

NC = 2
NS = 16
NW = NC * NS


def _mm_body(x_ref, w_ref, y_ref):
    y_ref[...] = jnp.dot(x_ref[...], w_ref[0], preferred_element_type=jnp.float32)


def _transform_features(x, W):
    n, c_in = x.shape
    k, _, c_out = W.shape
    return pl.pallas_call(
        _mm_body,
        grid=(k,),
        in_specs=[
            pl.BlockSpec((n, c_in), lambda i: (0, 0)),
            pl.BlockSpec((1, c_in, c_out), lambda i: (i, 0, 0)),
        ],
        out_specs=pl.BlockSpec((n, c_out), lambda i: (i, 0)),
        out_shape=jax.ShapeDtypeStruct((k * n, c_out), jnp.float32),
    )(x, W)


def _edge_prep_body(n, ei_ref, ko_ref, g_ref, d_ref):
    g_ref[...] = ko_ref[...] * n + ei_ref[0, :]
    d_ref[...] = ei_ref[1, :]


def _edge_prep(edge_index, kernel_offsets, n):
    e = edge_index.shape[1]
    return pl.pallas_call(
        functools.partial(_edge_prep_body, n),
        in_specs=[
            pl.BlockSpec((2, e), lambda: (0, 0)),
            pl.BlockSpec((e,), lambda: (0,)),
        ],
        out_specs=[
            pl.BlockSpec((e,), lambda: (0,)),
            pl.BlockSpec((e,), lambda: (0,)),
        ],
        out_shape=[
            jax.ShapeDtypeStruct((e,), jnp.int32),
            jax.ShapeDtypeStruct((e,), jnp.int32),
        ],
    )(edge_index, kernel_offsets)


def _make_sc_edge_kernel(n, c, e, ch, b):
    mesh = plsc.VectorSubcoreMesh(
        core_axis_name="c", subcore_axis_name="s", num_cores=NC, num_subcores=NS
    )
    r8 = (n // NS) // 8 * 8
    r_last = n - (NS - 1) * r8
    e_per_w = e // NW

    @functools.partial(
        pl.kernel,
        mesh=mesh,
        out_type=jax.ShapeDtypeStruct((NC, n, c), jnp.float32),
        scratch_types=[
            pltpu.VMEM_SHARED((n, c), jnp.float32),
            pltpu.VMEM((e_per_w,), jnp.int32),
            pltpu.VMEM((b,), jnp.int32),
            pltpu.VMEM((b,), jnp.int32),
            pltpu.VMEM((b,), jnp.int32),
            pltpu.VMEM((b, c), jnp.float32),
            pltpu.VMEM((b, c), jnp.float32),
            pltpu.VMEM((b, c), jnp.float32),
            pltpu.SemaphoreType.DMA,
            pltpu.SemaphoreType.DMA,
            pltpu.SemaphoreType.DMA,
            pltpu.SemaphoreType.DMA,
            pltpu.SemaphoreType.DMA,
            pltpu.SemaphoreType.DMA,
            pltpu.SemaphoreType.DMA,
            pltpu.SemaphoreType.DMA,
            pltpu.SemaphoreType.DMA,
        ],
    )
    def sc_kernel(y_hbm, gidx_hbm, dst_hbm, zeros_hbm, part_hbm,
                  acc, gidx_v, dst_a, dst_b, dst_c, rows_a, rows_b, rows_c,
                  sem_ra, sem_rb, sem_rc, sem_da, sem_db, sem_dc,
                  sem_sa, sem_sb, sem_sc):
        cid = lax.axis_index("c")
        sid = lax.axis_index("s")
        wid = cid * NS + sid
        base = wid * e_per_w

        pltpu.sync_copy(gidx_hbm.at[pl.ds(base, e_per_w)], gidx_v)

        def chunk(g, dst_buf, rows_buf, sem_r, sem_d):
            d_d = pltpu.async_copy(dst_hbm.at[pl.ds(base + g * b, b)], dst_buf, sem_d)
            d_r = pltpu.async_copy(y_hbm.at[gidx_v.at[pl.ds(g * b, b)]], rows_buf, sem_r)
            return d_d, d_r

        def drain(pair):
            d_d, d_r = pair
            d_r.wait()
            d_d.wait()

        in_a0 = chunk(0, dst_a, rows_a, sem_ra, sem_da)
        in_b0 = chunk(1, dst_b, rows_b, sem_rb, sem_db)
        in_c0 = chunk(2, dst_c, rows_c, sem_rc, sem_dc)

        @pl.when(sid < NS - 1)
        def _():
            pltpu.sync_copy(zeros_hbm.at[pl.ds(0, r8)], acc.at[pl.ds(sid * r8, r8)])

        @pl.when(sid == NS - 1)
        def _():
            pltpu.sync_copy(zeros_hbm, acc.at[pl.ds((NS - 1) * r8, r_last)])

        plsc.subcore_barrier()

        def scatter_triple(in_a, in_b, in_c):
            drain(in_a)
            s_a = pltpu.async_copy(rows_a, acc.at[dst_a], sem_sa, add=True)
            drain(in_b)
            s_b = pltpu.async_copy(rows_b, acc.at[dst_b], sem_sb, add=True)
            drain(in_c)
            s_c = pltpu.async_copy(rows_c, acc.at[dst_c], sem_sc, add=True)
            s_a.wait()
            s_b.wait()
            s_c.wait()

        scatter_triple(in_a0, in_b0, in_c0)

        def triple_body(p, carry):
            g = p * 3
            in_a = chunk(g, dst_a, rows_a, sem_ra, sem_da)
            in_b = chunk(g + 1, dst_b, rows_b, sem_rb, sem_db)
            in_c = chunk(g + 2, dst_c, rows_c, sem_rc, sem_dc)
            scatter_triple(in_a, in_b, in_c)
            return carry

        lax.fori_loop(1, ch // 3, triple_body, 0)
        for g in range((ch // 3) * 3, ch):
            in_t = chunk(g, dst_a, rows_a, sem_ra, sem_da)
            drain(in_t)
            pltpu.sync_copy(rows_a, acc.at[dst_a], add=True)
        plsc.subcore_barrier()

        @pl.when(sid < NS - 1)
        def _():
            pltpu.sync_copy(
                acc.at[pl.ds(sid * r8, r8)],
                part_hbm.at[cid].at[pl.ds(sid * r8, r8)],
            )

        @pl.when(sid == NS - 1)
        def _():
            pltpu.sync_copy(
                acc.at[pl.ds((NS - 1) * r8, r_last)],
                part_hbm.at[cid].at[pl.ds((NS - 1) * r8, r_last)],
            )

    return sc_kernel


def _combine_bn_body(n, p_ref, g_ref, b_ref, o_ref):
    s = p_ref[0] + p_ref[1]
    mean = jnp.sum(s, axis=0, keepdims=True) / n
    var = jnp.sum(s * s, axis=0, keepdims=True) / n - mean * mean
    scale = lax.rsqrt(var + BN_EPS) * g_ref[...]
    o_ref[...] = (s - mean) * scale + b_ref[...]


def _combine_bn(part, gamma, beta):
    _, n, c = part.shape
    return pl.pallas_call(
        functools.partial(_combine_bn_body, n),
        in_specs=[
            pl.BlockSpec((2, n, c), lambda: (0, 0, 0)),
            pl.BlockSpec((1, c), lambda: (0, 0)),
            pl.BlockSpec((1, c), lambda: (0, 0)),
        ],
        out_specs=pl.BlockSpec((n, c), lambda: (0, 0)),
        out_shape=jax.ShapeDtypeStruct((n, c), jnp.float32),
    )(part, gamma, beta)


@jax.jit
def kernel(x, edge_index, kernel_offsets, W, bn_gamma, bn_beta):
    n, c_in = x.shape
    k, _, c_out = W.shape
    e = edge_index.shape[1]

    y = _transform_features(x, W)

    gidx, dst = _edge_prep(edge_index, kernel_offsets, n)

    b = 80
    ch = e // (NW * b)
    r8 = (n // NS) // 8 * 8
    zeros = jnp.zeros((n - (NS - 1) * r8, c_out), jnp.float32)

    part = _make_sc_edge_kernel(n, c_out, e, ch, b)(y, gidx, dst, zeros)

    return _combine_bn(part, bn_gamma.reshape(1, c_out),
                       bn_beta.reshape(1, c_out))
